# Initial kernel scaffold; baseline (speedup 1.0000x reference)
import jax
import jax.numpy as jnp
from jax.experimental import pallas as pl


def kernel(inputs_row, target_row, target, center):
    raise NotImplementedError("write your pallas kernel here")


# speedup vs baseline: 1.7074x; 1.7074x over previous
import functools

import jax
import jax.numpy as jnp
from jax import lax
from jax.experimental import pallas as pl
from jax.experimental.pallas import tpu as pltpu
from jax.experimental.pallas import tpu_sc as plsc

NUM_CLASSES = 100000
EMB = 64
N = 16384
T = 1024

RB = 1024
NSTEP = N // RB
NC = 2
NS = 16
NW = NC * NS
CH = 400
NCHUNK = NUM_CLASSES // CH
KMAX = -(-NCHUNK // NW)


def _stats_body(tr_ref, tgt_ref, x_ref, meansT_ref, tidx_ref, cnt_ref):
    i = pl.program_id(0)

    @pl.when(i == 0)
    def _init():
        meansT_ref[...] = jnp.zeros_like(meansT_ref)
        cnt_ref[...] = jnp.zeros_like(cnt_ref)

    tr = tr_ref[0, 0, :]
    tgt = tgt_ref[0, :]
    oh = (tr[:, None] == tgt[None, :]).astype(jnp.float32)
    x = x_ref[...]
    meansT_ref[...] += lax.dot_general(
        x, oh, (((0,), (0,)), ((), ())), preferred_element_type=jnp.float32)
    ones = jnp.ones((1, RB), dtype=jnp.float32)
    cnt_ref[0:1, :] += lax.dot_general(
        ones, oh, (((1,), (0,)), ((), ())), preferred_element_type=jnp.float32)

    @pl.when(i == NSTEP - 1)
    def _fin():
        c = cnt_ref[0:1, :]
        meansT_ref[...] = meansT_ref[...] / jnp.maximum(c, 1.0)
        tidx_ref[...] = jnp.where(c > 0.0, tgt_ref[...], -1)


def _stats(inputs_row, target_row, target):
    tr3 = target_row.reshape(NSTEP, 1, RB)
    tgt2 = target.reshape(1, T)
    meansT, tidx = pl.pallas_call(
        _stats_body,
        grid=(NSTEP,),
        in_specs=[
            pl.BlockSpec((1, 1, RB), lambda i: (i, 0, 0)),
            pl.BlockSpec((1, T), lambda i: (0, 0)),
            pl.BlockSpec((RB, EMB), lambda i: (i, 0)),
        ],
        out_specs=[
            pl.BlockSpec((EMB, T), lambda i: (0, 0)),
            pl.BlockSpec((1, T), lambda i: (0, 0)),
        ],
        out_shape=[
            jax.ShapeDtypeStruct((EMB, T), jnp.float32),
            jax.ShapeDtypeStruct((1, T), jnp.int32),
        ],
        scratch_shapes=[pltpu.VMEM((8, T), jnp.float32)],
    )(tr3, tgt2, inputs_row)
    return meansT, tidx


def _sc_body(center_hbm, meansT_hbm, tidx_hbm, out_hbm, buf, mT, tid):
    wid = lax.axis_index("s") * NC + lax.axis_index("c")
    pltpu.sync_copy(meansT_hbm, mT)
    pltpu.sync_copy(tidx_hbm, tid)

    def round_body(k, carry):
        ci = wid + k * NW

        @pl.when(ci < NCHUNK)
        def _do_chunk():
            cbase = ci * CH
            pltpu.sync_copy(center_hbm.at[pl.ds(cbase, CH)], buf)

            def group_body(g, carry2):
                tvec = tid[pl.ds(g * 16, 16)]
                local = tvec - cbase
                mask = (local >= 0) & (local < CH)

                @pl.when(jnp.any(mask))
                def _patch():
                    safe = jnp.where(mask, local, 0)

                    def col_body(cc, carry3):
                        val = mT[pl.ds(cc * T + g * 16, 16)]
                        col = jnp.full((16,), cc, dtype=jnp.int32)
                        plsc.store_scatter(buf, [safe, col], val, mask=mask)
                        return carry3

                    lax.fori_loop(0, EMB, col_body, 0, unroll=False)
                return carry2

            lax.fori_loop(0, T // 16, group_body, 0, unroll=False)
            pltpu.sync_copy(buf, out_hbm.at[pl.ds(cbase, CH)])
        return carry

    lax.fori_loop(0, KMAX, round_body, 0, unroll=False)


@functools.cache
def _sc_scatter():
    return pl.kernel(
        _sc_body,
        out_type=jax.ShapeDtypeStruct((NUM_CLASSES, EMB), jnp.float32),
        mesh=plsc.VectorSubcoreMesh(core_axis_name="c", subcore_axis_name="s"),
        scratch_types=[
            pltpu.VMEM((CH, EMB), jnp.float32),
            pltpu.VMEM((EMB * T,), jnp.float32),
            pltpu.VMEM((T,), jnp.int32),
        ],
        compiler_params=pltpu.CompilerParams(needs_layout_passes=False),
    )


def kernel(inputs_row, target_row, target, center):
    meansT, tidx = _stats(inputs_row, target_row, target)
    return _sc_scatter()(center, meansT.reshape(EMB * T), tidx.reshape(T))


# speedup vs baseline: 3.6021x; 2.1097x over previous
import functools

import jax
import jax.numpy as jnp
from jax import lax
from jax.experimental import pallas as pl
from jax.experimental.pallas import tpu as pltpu
from jax.experimental.pallas import tpu_sc as plsc

NUM_CLASSES = 100000
EMB = 64
N = 16384
T = 1024

RB = 1024
NSTEP = N // RB
NC = 2
NS = 16
NW = NC * NS
CHC = 384
NALIGN = (NUM_CLASSES // 128) * 128
NFULL = NALIGN // CHC
MIDC = NALIGN - NFULL * CHC
TAILC = NUM_CLASSES - NALIGN
NCHUNK = NFULL + 1
KMAX = -(-NCHUNK // NW)


def _stats_body(tr_ref, tgt_ref, xT_ref, ctail_ref,
                meansT_ref, tidx_ref, tail_ref, cnt_ref):
    i = pl.program_id(0)

    @pl.when(i == 0)
    def _init():
        meansT_ref[...] = jnp.zeros_like(meansT_ref)
        cnt_ref[...] = jnp.zeros_like(cnt_ref)

    tr = tr_ref[0, 0, :]
    tgt = tgt_ref[0, :]
    oh = (tr[:, None] == tgt[None, :]).astype(jnp.float32)
    xT = xT_ref[...]
    meansT_ref[...] += lax.dot_general(
        xT, oh, (((1,), (0,)), ((), ())), preferred_element_type=jnp.float32)
    ones = jnp.ones((1, RB), dtype=jnp.float32)
    cnt_ref[0:1, :] += lax.dot_general(
        ones, oh, (((1,), (0,)), ((), ())), preferred_element_type=jnp.float32)

    @pl.when(i == NSTEP - 1)
    def _fin():
        c = cnt_ref[0:1, :]
        meansT_ref[...] = meansT_ref[...] / jnp.maximum(c, 1.0)
        tidx_ref[...] = jnp.where(c > 0.0, tgt_ref[...], -1)
        tcls = NALIGN + lax.broadcasted_iota(jnp.int32, (TAILC, 1), 0)
        m = (tcls == tgt[None, :]).astype(jnp.float32)
        msum = lax.dot_general(
            meansT_ref[...], m, (((1,), (1,)), ((), ())),
            preferred_element_type=jnp.float32)
        dup = lax.dot_general(
            ones[:, :T], m, (((1,), (1,)), ((), ())),
            preferred_element_type=jnp.float32)
        csum = lax.dot_general(
            c, m, (((1,), (1,)), ((), ())),
            preferred_element_type=jnp.float32)
        upd = (dup > 0.0) & (csum > 0.0)
        tail_ref[...] = jnp.where(
            upd, msum / jnp.maximum(dup, 1.0), ctail_ref[...])


def _stats(xT, target_row, target, ctailT):
    tr3 = target_row.reshape(NSTEP, 1, RB)
    tgt2 = target.reshape(1, T)
    ones_spec = pl.BlockSpec((EMB, TAILC), lambda i: (0, 0))
    meansT, tidx, tail = pl.pallas_call(
        _stats_body,
        grid=(NSTEP,),
        in_specs=[
            pl.BlockSpec((1, 1, RB), lambda i: (i, 0, 0)),
            pl.BlockSpec((1, T), lambda i: (0, 0)),
            pl.BlockSpec((EMB, RB), lambda i: (0, i)),
            ones_spec,
        ],
        out_specs=[
            pl.BlockSpec((EMB, T), lambda i: (0, 0)),
            pl.BlockSpec((1, T), lambda i: (0, 0)),
            ones_spec,
        ],
        out_shape=[
            jax.ShapeDtypeStruct((EMB, T), jnp.float32),
            jax.ShapeDtypeStruct((1, T), jnp.int32),
            jax.ShapeDtypeStruct((EMB, TAILC), jnp.float32),
        ],
        scratch_shapes=[pltpu.VMEM((8, T), jnp.float32)],
    )(tr3, tgt2, xT, ctailT)
    return meansT, tidx, tail


def _patch_cols(buf, mT, tid, cbase, ncols):
    def group_body(g, carry):
        tvec = tid[0, pl.ds(g * 16, 16)]
        local = tvec - cbase
        mask = (local >= 0) & (local < ncols)

        @pl.when(jnp.any(mask))
        def _patch():
            safe = jnp.where(mask, local, 0)

            def row_body(r, carry3):
                val = mT[r, pl.ds(g * 16, 16)]
                row = jnp.full((16,), r, dtype=jnp.int32)
                plsc.store_scatter(buf, [row, safe], val, mask=mask)
                return carry3

            lax.fori_loop(0, EMB, row_body, 0, unroll=False)
        return carry

    lax.fori_loop(0, T // 16, group_body, 0, unroll=False)


def _sc_body(centerT_hbm, meansT_hbm, tidx_hbm, outT_hbm, buf, mT, tid):
    wid = lax.axis_index("s") * NC + lax.axis_index("c")
    pltpu.sync_copy(meansT_hbm, mT)
    pltpu.sync_copy(tidx_hbm, tid)

    def round_body(k, carry):
        ci = wid + k * NW

        @pl.when(ci < NFULL)
        def _full_chunk():
            cbase = ci * CHC
            pltpu.sync_copy(centerT_hbm.at[:, pl.ds(cbase, CHC)], buf)
            _patch_cols(buf, mT, tid, cbase, CHC)
            pltpu.sync_copy(buf, outT_hbm.at[:, pl.ds(cbase, CHC)])

        @pl.when(ci == NFULL)
        def _mid_chunk():
            cbase = NFULL * CHC
            mbuf = buf.at[:, pl.ds(0, MIDC)]
            pltpu.sync_copy(centerT_hbm.at[:, pl.ds(cbase, MIDC)], mbuf)
            _patch_cols(buf, mT, tid, cbase, MIDC)
            pltpu.sync_copy(mbuf, outT_hbm.at[:, pl.ds(cbase, MIDC)])
        return carry

    lax.fori_loop(0, KMAX, round_body, 0, unroll=False)


@functools.cache
def _sc_scatter():
    return pl.kernel(
        _sc_body,
        out_type=jax.ShapeDtypeStruct((EMB, NUM_CLASSES), jnp.float32),
        mesh=plsc.VectorSubcoreMesh(core_axis_name="c", subcore_axis_name="s"),
        scratch_types=[
            pltpu.VMEM((EMB, CHC), jnp.float32),
            pltpu.VMEM((EMB, T), jnp.float32),
            pltpu.VMEM((1, T), jnp.int32),
        ],
        compiler_params=pltpu.CompilerParams(needs_layout_passes=False),
    )


def kernel(inputs_row, target_row, target, center):
    centerT = center.T
    meansT, tidx, tail = _stats(
        inputs_row.T, target_row, target, centerT[:, NALIGN:])
    outT = _sc_scatter()(centerT, meansT, tidx)
    outT = lax.dynamic_update_slice(outT, tail, (0, NALIGN))
    return outT.T


# speedup vs baseline: 3.9253x; 1.0897x over previous
import functools

import jax
import jax.numpy as jnp
from jax import lax
from jax.experimental import pallas as pl
from jax.experimental.pallas import tpu as pltpu
from jax.experimental.pallas import tpu_sc as plsc

NUM_CLASSES = 100000
EMB = 64
N = 16384
T = 1024

RB = 1024
NSTEP = N // RB
NC = 2
NS = 16
NW = NC * NS
CHC = 256
NALIGN = (NUM_CLASSES // 128) * 128
NFULL = NALIGN // CHC
MIDC = NALIGN - NFULL * CHC
TAILC = NUM_CLASSES - NALIGN
NCHUNK = NFULL + 1
KMAX = -(-NCHUNK // NW)


def _stats_body(tr_ref, tgt_ref, xT_ref, ctail_ref,
                meansT_ref, tidx_ref, tail_ref, cnt_ref):
    i = pl.program_id(0)

    @pl.when(i == 0)
    def _init():
        meansT_ref[...] = jnp.zeros_like(meansT_ref)
        cnt_ref[...] = jnp.zeros_like(cnt_ref)

    tr = tr_ref[0, 0, :]
    tgt = tgt_ref[0, :]
    oh = (tr[:, None] == tgt[None, :]).astype(jnp.float32)
    xT = xT_ref[...]
    meansT_ref[...] += lax.dot_general(
        xT, oh, (((1,), (0,)), ((), ())), preferred_element_type=jnp.float32)
    ones = jnp.ones((1, RB), dtype=jnp.float32)
    cnt_ref[0:1, :] += lax.dot_general(
        ones, oh, (((1,), (0,)), ((), ())), preferred_element_type=jnp.float32)

    @pl.when(i == NSTEP - 1)
    def _fin():
        c = cnt_ref[0:1, :]
        meansT_ref[...] = meansT_ref[...] / jnp.maximum(c, 1.0)
        tidx_ref[...] = jnp.where(c > 0.0, tgt_ref[...], -1)
        tcls = NALIGN + lax.broadcasted_iota(jnp.int32, (TAILC, 1), 0)
        m = (tcls == tgt[None, :]).astype(jnp.float32)
        msum = lax.dot_general(
            meansT_ref[...], m, (((1,), (1,)), ((), ())),
            preferred_element_type=jnp.float32)
        dup = lax.dot_general(
            ones[:, :T], m, (((1,), (1,)), ((), ())),
            preferred_element_type=jnp.float32)
        csum = lax.dot_general(
            c, m, (((1,), (1,)), ((), ())),
            preferred_element_type=jnp.float32)
        upd = (dup > 0.0) & (csum > 0.0)
        tail_ref[...] = jnp.where(
            upd, msum / jnp.maximum(dup, 1.0), ctail_ref[...])


def _stats(xT, target_row, target, ctailT):
    tr3 = target_row.reshape(NSTEP, 1, RB)
    tgt2 = target.reshape(1, T)
    ones_spec = pl.BlockSpec((EMB, TAILC), lambda i: (0, 0))
    meansT, tidx, tail = pl.pallas_call(
        _stats_body,
        grid=(NSTEP,),
        in_specs=[
            pl.BlockSpec((1, 1, RB), lambda i: (i, 0, 0)),
            pl.BlockSpec((1, T), lambda i: (0, 0)),
            pl.BlockSpec((EMB, RB), lambda i: (0, i)),
            ones_spec,
        ],
        out_specs=[
            pl.BlockSpec((EMB, T), lambda i: (0, 0)),
            pl.BlockSpec((1, T), lambda i: (0, 0)),
            ones_spec,
        ],
        out_shape=[
            jax.ShapeDtypeStruct((EMB, T), jnp.float32),
            jax.ShapeDtypeStruct((1, T), jnp.int32),
            jax.ShapeDtypeStruct((EMB, TAILC), jnp.float32),
        ],
        scratch_shapes=[pltpu.VMEM((8, T), jnp.float32)],
    )(tr3, tgt2, xT, ctailT)
    return meansT, tidx, tail


def _patch_cols(buf, mT, tid, cbase, ncols):
    def group_body(g, carry):
        tvec = tid[0, pl.ds(g * 16, 16)]
        local = tvec - cbase
        mask = (local >= 0) & (local < ncols)

        @pl.when(jnp.any(mask))
        def _patch():
            safe = jnp.where(mask, local, 0)

            def row_body(r, carry3):
                val = mT[r, pl.ds(g * 16, 16)]
                row = jnp.full((16,), r, dtype=jnp.int32)
                plsc.store_scatter(buf, [row, safe], val, mask=mask)
                return carry3

            lax.fori_loop(0, EMB, row_body, 0, unroll=False)
        return carry

    lax.fori_loop(0, T // 16, group_body, 0, unroll=False)


def _sc_body(centerT_hbm, meansT_hbm, tidx_hbm, outT_hbm,
             buf0, buf1, mT, tid, sr0, sr1, sw0, sw1):
    wid = lax.axis_index("s") * NC + lax.axis_index("c")
    pltpu.sync_copy(meansT_hbm, mT)
    pltpu.sync_copy(tidx_hbm, tid)
    bufs = (buf0, buf1)
    srs = (sr0, sr1)
    sws = (sw0, sw1)

    def read_desc(ci, buf, sem):
        full = pltpu.make_async_copy(
            centerT_hbm.at[:, pl.ds(ci * CHC, CHC)], buf, sem)
        mid = pltpu.make_async_copy(
            centerT_hbm.at[:, pl.ds(NFULL * CHC, MIDC)],
            buf.at[:, pl.ds(0, MIDC)], sem)
        return full, mid

    def write_desc(ci, buf, sem):
        full = pltpu.make_async_copy(
            buf, outT_hbm.at[:, pl.ds(ci * CHC, CHC)], sem)
        mid = pltpu.make_async_copy(
            buf.at[:, pl.ds(0, MIDC)],
            outT_hbm.at[:, pl.ds(NFULL * CHC, MIDC)], sem)
        return full, mid

    def start_read(k, b):
        ci = wid + k * NW
        full, mid = read_desc(ci, bufs[b], srs[b])
        pl.when(ci < NFULL)(full.start)
        pl.when(ci == NFULL)(mid.start)

    def wait_read(k, b):
        ci = wid + k * NW
        full, mid = read_desc(ci, bufs[b], srs[b])
        pl.when(ci < NFULL)(full.wait)
        pl.when(ci == NFULL)(mid.wait)

    def start_write(k, b):
        ci = wid + k * NW
        full, mid = write_desc(ci, bufs[b], sws[b])
        pl.when(ci < NFULL)(full.start)
        pl.when(ci == NFULL)(mid.start)

    def wait_write(k, b):
        ci = wid + k * NW
        full, mid = write_desc(ci, bufs[b], sws[b])
        pl.when((ci >= 0) & (ci < NFULL))(full.wait)
        pl.when(ci == NFULL)(mid.wait)

    def patch(k, b):
        ci = wid + k * NW
        cbase = ci * CHC

        @pl.when(ci < NFULL)
        def _pf():
            _patch_cols(bufs[b], mT, tid, cbase, CHC)

        @pl.when(ci == NFULL)
        def _pm():
            _patch_cols(bufs[b], mT, tid, NFULL * CHC, MIDC)

    start_read(0, 0)

    def pair_body(k2, carry):
        for b in (0, 1):
            k = k2 * 2 + b
            wait_read(k, b)
            wait_write(k - 1, 1 - b)
            start_read(k + 1, 1 - b)
            patch(k, b)
            start_write(k, b)
        return carry

    lax.fori_loop(0, (KMAX + 2) // 2, pair_body, 0, unroll=False)


@functools.cache
def _sc_scatter():
    return pl.kernel(
        _sc_body,
        out_type=jax.ShapeDtypeStruct((EMB, NUM_CLASSES), jnp.float32),
        mesh=plsc.VectorSubcoreMesh(core_axis_name="c", subcore_axis_name="s"),
        scratch_types=[
            pltpu.VMEM((EMB, CHC), jnp.float32),
            pltpu.VMEM((EMB, CHC), jnp.float32),
            pltpu.VMEM((EMB, T), jnp.float32),
            pltpu.VMEM((1, T), jnp.int32),
            pltpu.SemaphoreType.DMA,
            pltpu.SemaphoreType.DMA,
            pltpu.SemaphoreType.DMA,
            pltpu.SemaphoreType.DMA,
        ],
        compiler_params=pltpu.CompilerParams(needs_layout_passes=False),
    )


def kernel(inputs_row, target_row, target, center):
    centerT = center.T
    meansT, tidx, tail = _stats(
        inputs_row.T, target_row, target, centerT[:, NALIGN:])
    outT = _sc_scatter()(centerT, meansT, tidx)
    outT = lax.dynamic_update_slice(outT, tail, (0, NALIGN))
    return outT.T


# speedup vs baseline: 4.2598x; 1.0852x over previous
import functools

import jax
import jax.numpy as jnp
from jax import lax
from jax.experimental import pallas as pl
from jax.experimental.pallas import tpu as pltpu
from jax.experimental.pallas import tpu_sc as plsc

NUM_CLASSES = 100000
EMB = 64
N = 16384
T = 1024

RB = 1024
NSTEP = N // RB
NC = 2
NS = 16
NW = NC * NS
CHC = 384
NALIGN = (NUM_CLASSES // 128) * 128
NFULL = NALIGN // CHC
MIDC = NALIGN - NFULL * CHC
TAILC = NUM_CLASSES - NALIGN
NCHUNK = NFULL + 1
KMAX = -(-NCHUNK // NW)


def _stats_body(tr_ref, tgt_ref, xT_ref, ctail_ref,
                meansT_ref, tidx_ref, tail_ref, cnt_ref):
    i = pl.program_id(0)

    @pl.when(i == 0)
    def _init():
        meansT_ref[...] = jnp.zeros_like(meansT_ref)
        cnt_ref[...] = jnp.zeros_like(cnt_ref)

    tr = tr_ref[0, 0, :]
    tgt = tgt_ref[0, :]
    oh = (tr[:, None] == tgt[None, :]).astype(jnp.bfloat16)
    xT = xT_ref[...].astype(jnp.bfloat16)
    meansT_ref[...] += lax.dot_general(
        xT, oh, (((1,), (0,)), ((), ())), preferred_element_type=jnp.float32)
    ones = jnp.ones((1, RB), dtype=jnp.bfloat16)
    cnt_ref[0:1, :] += lax.dot_general(
        ones, oh, (((1,), (0,)), ((), ())), preferred_element_type=jnp.float32)

    @pl.when(i == NSTEP - 1)
    def _fin():
        c = cnt_ref[0:1, :]
        meansT_ref[...] = meansT_ref[...] / jnp.maximum(c, 1.0)
        tidx_ref[...] = jnp.where(c > 0.0, tgt_ref[...], -1)
        tcls = NALIGN + lax.broadcasted_iota(jnp.int32, (TAILC, 1), 0)
        m = (tcls == tgt[None, :]).astype(jnp.float32)
        msum = lax.dot_general(
            meansT_ref[...], m, (((1,), (1,)), ((), ())),
            preferred_element_type=jnp.float32)
        dup = lax.dot_general(
            jnp.ones((1, T), dtype=jnp.float32), m, (((1,), (1,)), ((), ())),
            preferred_element_type=jnp.float32)
        csum = lax.dot_general(
            c, m, (((1,), (1,)), ((), ())),
            preferred_element_type=jnp.float32)
        upd = (dup > 0.0) & (csum > 0.0)
        tail_ref[...] = jnp.where(
            upd, msum / jnp.maximum(dup, 1.0), ctail_ref[...])


def _stats(xT, target_row, target, ctailT):
    tr3 = target_row.reshape(NSTEP, 1, RB)
    tgt2 = target.reshape(1, T)
    ones_spec = pl.BlockSpec((EMB, TAILC), lambda i: (0, 0))
    meansT, tidx, tail = pl.pallas_call(
        _stats_body,
        grid=(NSTEP,),
        in_specs=[
            pl.BlockSpec((1, 1, RB), lambda i: (i, 0, 0)),
            pl.BlockSpec((1, T), lambda i: (0, 0)),
            pl.BlockSpec((EMB, RB), lambda i: (0, i)),
            ones_spec,
        ],
        out_specs=[
            pl.BlockSpec((EMB, T), lambda i: (0, 0)),
            pl.BlockSpec((1, T), lambda i: (0, 0)),
            ones_spec,
        ],
        out_shape=[
            jax.ShapeDtypeStruct((EMB, T), jnp.float32),
            jax.ShapeDtypeStruct((1, T), jnp.int32),
            jax.ShapeDtypeStruct((EMB, TAILC), jnp.float32),
        ],
        scratch_shapes=[pltpu.VMEM((8, T), jnp.float32)],
    )(tr3, tgt2, xT, ctailT)
    return meansT, tidx, tail


def _patch_cols(buf, mT, tid, cbase, ncols):
    def group_body(g, carry):
        tvec = tid[0, pl.ds(g * 16, 16)]
        local = tvec - cbase
        mask = (local >= 0) & (local < ncols)

        @pl.when(jnp.any(mask))
        def _patch():
            safe = jnp.where(mask, local, 0)

            def row_body(r, carry3):
                val = mT[r, pl.ds(g * 16, 16)]
                row = jnp.full((16,), r, dtype=jnp.int32)
                plsc.store_scatter(buf, [row, safe], val, mask=mask)
                return carry3

            lax.fori_loop(0, EMB, row_body, 0, unroll=False)
        return carry

    lax.fori_loop(0, T // 16, group_body, 0, unroll=False)


def _sc_body(centerT_hbm, meansT_hbm, tidx_hbm, outT_hbm,
             buf0, buf1, mT, tid, sr0, sr1, sw0, sw1):
    wid = lax.axis_index("s") * NC + lax.axis_index("c")
    pltpu.sync_copy(meansT_hbm, mT)
    pltpu.sync_copy(tidx_hbm, tid)
    bufs = (buf0, buf1)
    srs = (sr0, sr1)
    sws = (sw0, sw1)

    def read_desc(ci, buf, sem):
        full = pltpu.make_async_copy(
            centerT_hbm.at[:, pl.ds(ci * CHC, CHC)], buf, sem)
        mid = pltpu.make_async_copy(
            centerT_hbm.at[:, pl.ds(NFULL * CHC, MIDC)],
            buf.at[:, pl.ds(0, MIDC)], sem)
        return full, mid

    def write_desc(ci, buf, sem):
        full = pltpu.make_async_copy(
            buf, outT_hbm.at[:, pl.ds(ci * CHC, CHC)], sem)
        mid = pltpu.make_async_copy(
            buf.at[:, pl.ds(0, MIDC)],
            outT_hbm.at[:, pl.ds(NFULL * CHC, MIDC)], sem)
        return full, mid

    def start_read(k, b):
        ci = wid + k * NW
        full, mid = read_desc(ci, bufs[b], srs[b])
        pl.when(ci < NFULL)(full.start)
        pl.when(ci == NFULL)(mid.start)

    def wait_read(k, b):
        ci = wid + k * NW
        full, mid = read_desc(ci, bufs[b], srs[b])
        pl.when(ci < NFULL)(full.wait)
        pl.when(ci == NFULL)(mid.wait)

    def start_write(k, b):
        ci = wid + k * NW
        full, mid = write_desc(ci, bufs[b], sws[b])
        pl.when(ci < NFULL)(full.start)
        pl.when(ci == NFULL)(mid.start)

    def wait_write(k, b):
        ci = wid + k * NW
        full, mid = write_desc(ci, bufs[b], sws[b])
        pl.when((ci >= 0) & (ci < NFULL))(full.wait)
        pl.when(ci == NFULL)(mid.wait)

    def patch(k, b):
        ci = wid + k * NW
        cbase = ci * CHC

        @pl.when(ci < NFULL)
        def _pf():
            _patch_cols(bufs[b], mT, tid, cbase, CHC)

        @pl.when(ci == NFULL)
        def _pm():
            _patch_cols(bufs[b], mT, tid, NFULL * CHC, MIDC)

    start_read(0, 0)

    def pair_body(k2, carry):
        for b in (0, 1):
            k = k2 * 2 + b
            wait_read(k, b)
            wait_write(k - 1, 1 - b)
            start_read(k + 1, 1 - b)
            patch(k, b)
            start_write(k, b)
        return carry

    lax.fori_loop(0, (KMAX + 2) // 2, pair_body, 0, unroll=False)


@functools.cache
def _sc_scatter():
    return pl.kernel(
        _sc_body,
        out_type=jax.ShapeDtypeStruct((EMB, NUM_CLASSES), jnp.float32),
        mesh=plsc.VectorSubcoreMesh(core_axis_name="c", subcore_axis_name="s"),
        scratch_types=[
            pltpu.VMEM((EMB, CHC), jnp.float32),
            pltpu.VMEM((EMB, CHC), jnp.float32),
            pltpu.VMEM((EMB, T), jnp.float32),
            pltpu.VMEM((1, T), jnp.int32),
            pltpu.SemaphoreType.DMA,
            pltpu.SemaphoreType.DMA,
            pltpu.SemaphoreType.DMA,
            pltpu.SemaphoreType.DMA,
        ],
        compiler_params=pltpu.CompilerParams(needs_layout_passes=False),
    )


def kernel(inputs_row, target_row, target, center):
    centerT = center.T
    meansT, tidx, tail = _stats(
        inputs_row.T, target_row, target, centerT[:, NALIGN:])
    outT = _sc_scatter()(centerT, meansT, tidx)
    outT = lax.dynamic_update_slice(outT, tail, (0, NALIGN))
    return outT.T


# speedup vs baseline: 4.4110x; 1.0355x over previous
import functools

import jax
import jax.numpy as jnp
from jax import lax
from jax.experimental import pallas as pl
from jax.experimental.pallas import tpu as pltpu
from jax.experimental.pallas import tpu_sc as plsc

NUM_CLASSES = 100000
EMB = 64
N = 16384
T = 1024

RB = 1024
NSTEP = N // RB
NC = 2
NS = 16
NW = NC * NS
CHC = 384
NALIGN = (NUM_CLASSES // 128) * 128
NFULL = NALIGN // CHC
MIDC = NALIGN - NFULL * CHC
TAILC = NUM_CLASSES - NALIGN
NCHUNK = NFULL + 1
KMAX = -(-NCHUNK // NW)


def _stats_body(tr_ref, tgt_ref, xT_ref, ctail_ref,
                meansT_ref, tidx_ref, tail_ref, acc_ref):
    i = pl.program_id(0)

    @pl.when(i == 0)
    def _init():
        acc_ref[...] = jnp.zeros_like(acc_ref)

    tr = tr_ref[0, 0, :]
    tgt = tgt_ref[0, :]
    oh = (tr[:, None] == tgt[None, :]).astype(jnp.bfloat16)
    xT = xT_ref[...].astype(jnp.bfloat16)
    xa = jnp.concatenate(
        [xT, jnp.ones((8, RB), dtype=jnp.bfloat16)], axis=0)
    acc_ref[...] += lax.dot_general(
        xa, oh, (((1,), (0,)), ((), ())), preferred_element_type=jnp.float32)

    @pl.when(i == NSTEP - 1)
    def _fin():
        c = acc_ref[EMB:EMB + 1, :]
        meansT_ref[...] = acc_ref[0:EMB, :] / jnp.maximum(c, 1.0)
        tidx_ref[...] = jnp.where(c > 0.0, tgt_ref[...], -1)
        tcls = NALIGN + lax.broadcasted_iota(jnp.int32, (TAILC, 1), 0)
        m = (tcls == tgt[None, :]).astype(jnp.float32)
        msum = lax.dot_general(
            meansT_ref[...], m, (((1,), (1,)), ((), ())),
            preferred_element_type=jnp.float32)
        dup = lax.dot_general(
            jnp.ones((1, T), dtype=jnp.float32), m, (((1,), (1,)), ((), ())),
            preferred_element_type=jnp.float32)
        csum = lax.dot_general(
            c, m, (((1,), (1,)), ((), ())),
            preferred_element_type=jnp.float32)
        upd = (dup > 0.0) & (csum > 0.0)
        tail_ref[...] = jnp.where(
            upd, msum / jnp.maximum(dup, 1.0), ctail_ref[...])


def _stats(xT, target_row, target, ctailT):
    tr3 = target_row.reshape(NSTEP, 1, RB)
    tgt2 = target.reshape(1, T)
    ones_spec = pl.BlockSpec((EMB, TAILC), lambda i: (0, 0))
    meansT, tidx, tail = pl.pallas_call(
        _stats_body,
        grid=(NSTEP,),
        in_specs=[
            pl.BlockSpec((1, 1, RB), lambda i: (i, 0, 0)),
            pl.BlockSpec((1, T), lambda i: (0, 0)),
            pl.BlockSpec((EMB, RB), lambda i: (0, i)),
            ones_spec,
        ],
        out_specs=[
            pl.BlockSpec((EMB, T), lambda i: (0, 0)),
            pl.BlockSpec((1, T), lambda i: (0, 0)),
            ones_spec,
        ],
        out_shape=[
            jax.ShapeDtypeStruct((EMB, T), jnp.float32),
            jax.ShapeDtypeStruct((1, T), jnp.int32),
            jax.ShapeDtypeStruct((EMB, TAILC), jnp.float32),
        ],
        scratch_shapes=[pltpu.VMEM((EMB + 8, T), jnp.float32)],
    )(tr3, tgt2, xT, ctailT)
    return meansT, tidx, tail


def _patch_cols(buf, mT, tid, cbase, ncols):
    def group_body(g, carry):
        tvec = tid[0, pl.ds(g * 16, 16)]
        local = tvec - cbase
        mask = (local >= 0) & (local < ncols)

        @pl.when(jnp.any(mask))
        def _patch():
            safe = jnp.where(mask, local, 0)

            def row_body(r, carry3):
                val = mT[r, pl.ds(g * 16, 16)]
                row = jnp.full((16,), r, dtype=jnp.int32)
                plsc.store_scatter(buf, [row, safe], val, mask=mask)
                return carry3

            lax.fori_loop(0, EMB, row_body, 0, unroll=False)
        return carry

    lax.fori_loop(0, T // 16, group_body, 0, unroll=False)


def _sc_body(centerT_hbm, meansT_hbm, tidx_hbm, outT_hbm,
             buf0, buf1, mT, tid, sr0, sr1, sw0, sw1):
    wid = lax.axis_index("s") * NC + lax.axis_index("c")
    pltpu.sync_copy(meansT_hbm, mT)
    pltpu.sync_copy(tidx_hbm, tid)
    bufs = (buf0, buf1)
    srs = (sr0, sr1)
    sws = (sw0, sw1)

    def read_desc(ci, buf, sem):
        full = pltpu.make_async_copy(
            centerT_hbm.at[:, pl.ds(ci * CHC, CHC)], buf, sem)
        mid = pltpu.make_async_copy(
            centerT_hbm.at[:, pl.ds(NFULL * CHC, MIDC)],
            buf.at[:, pl.ds(0, MIDC)], sem)
        return full, mid

    def write_desc(ci, buf, sem):
        full = pltpu.make_async_copy(
            buf, outT_hbm.at[:, pl.ds(ci * CHC, CHC)], sem)
        mid = pltpu.make_async_copy(
            buf.at[:, pl.ds(0, MIDC)],
            outT_hbm.at[:, pl.ds(NFULL * CHC, MIDC)], sem)
        return full, mid

    def start_read(k, b):
        ci = wid + k * NW
        full, mid = read_desc(ci, bufs[b], srs[b])
        pl.when(ci < NFULL)(full.start)
        pl.when(ci == NFULL)(mid.start)

    def wait_read(k, b):
        ci = wid + k * NW
        full, mid = read_desc(ci, bufs[b], srs[b])
        pl.when(ci < NFULL)(full.wait)
        pl.when(ci == NFULL)(mid.wait)

    def start_write(k, b):
        ci = wid + k * NW
        full, mid = write_desc(ci, bufs[b], sws[b])
        pl.when(ci < NFULL)(full.start)
        pl.when(ci == NFULL)(mid.start)

    def wait_write(k, b):
        ci = wid + k * NW
        full, mid = write_desc(ci, bufs[b], sws[b])
        pl.when((ci >= 0) & (ci < NFULL))(full.wait)
        pl.when(ci == NFULL)(mid.wait)

    def patch(k, b):
        ci = wid + k * NW
        cbase = ci * CHC

        @pl.when(ci < NFULL)
        def _pf():
            _patch_cols(bufs[b], mT, tid, cbase, CHC)

        @pl.when(ci == NFULL)
        def _pm():
            _patch_cols(bufs[b], mT, tid, NFULL * CHC, MIDC)

    start_read(0, 0)

    def pair_body(k2, carry):
        for b in (0, 1):
            k = k2 * 2 + b
            wait_read(k, b)
            wait_write(k - 1, 1 - b)
            start_read(k + 1, 1 - b)
            patch(k, b)
            start_write(k, b)
        return carry

    lax.fori_loop(0, (KMAX + 2) // 2, pair_body, 0, unroll=False)


@functools.cache
def _sc_scatter():
    return pl.kernel(
        _sc_body,
        out_type=jax.ShapeDtypeStruct((EMB, NUM_CLASSES), jnp.float32),
        mesh=plsc.VectorSubcoreMesh(core_axis_name="c", subcore_axis_name="s"),
        scratch_types=[
            pltpu.VMEM((EMB, CHC), jnp.float32),
            pltpu.VMEM((EMB, CHC), jnp.float32),
            pltpu.VMEM((EMB, T), jnp.float32),
            pltpu.VMEM((1, T), jnp.int32),
            pltpu.SemaphoreType.DMA,
            pltpu.SemaphoreType.DMA,
            pltpu.SemaphoreType.DMA,
            pltpu.SemaphoreType.DMA,
        ],
        compiler_params=pltpu.CompilerParams(needs_layout_passes=False),
    )


def kernel(inputs_row, target_row, target, center):
    centerT = center.T
    meansT, tidx, tail = _stats(
        inputs_row.T, target_row, target, centerT[:, NALIGN:])
    outT = _sc_scatter()(centerT, meansT, tidx)
    outT = lax.dynamic_update_slice(outT, tail, (0, NALIGN))
    return outT.T


# speedup vs baseline: 4.5404x; 1.0293x over previous
import functools

import jax
import jax.numpy as jnp
from jax import lax
from jax.experimental import pallas as pl
from jax.experimental.pallas import tpu as pltpu
from jax.experimental.pallas import tpu_sc as plsc

NUM_CLASSES = 100000
EMB = 64
N = 16384
T = 1024

RB = 1024
NSTEP = N // RB
NC = 2
NS = 16
NW = NC * NS
CHC = 384
NALIGN = (NUM_CLASSES // 128) * 128
NFULL = NALIGN // CHC
MIDC = NALIGN - NFULL * CHC
TAILC = NUM_CLASSES - NALIGN
NCHUNK = NFULL + 1
KMAX = -(-NCHUNK // NW)


def _stats_body(tr_ref, tgt_ref, xT_ref, ctail_ref,
                meansT_ref, tidx_ref, tail_ref, acc_ref):
    i = pl.program_id(0)

    @pl.when(i == 0)
    def _init():
        acc_ref[...] = jnp.zeros_like(acc_ref)

    tr = tr_ref[0, 0, :]
    tgt = tgt_ref[0, :]
    oh = (tr[:, None] == tgt[None, :]).astype(jnp.bfloat16)
    xT = xT_ref[...].astype(jnp.bfloat16)
    xa = jnp.concatenate(
        [xT, jnp.ones((8, RB), dtype=jnp.bfloat16)], axis=0)
    acc_ref[...] += lax.dot_general(
        xa, oh, (((1,), (0,)), ((), ())), preferred_element_type=jnp.float32)

    @pl.when(i == NSTEP - 1)
    def _fin():
        c = acc_ref[EMB:EMB + 1, :]
        meansT_ref[...] = acc_ref[0:EMB, :] / jnp.maximum(c, 1.0)
        tidx_ref[...] = jnp.where(c > 0.0, tgt_ref[...], -1)
        tcls = NALIGN + lax.broadcasted_iota(jnp.int32, (TAILC, 1), 0)
        m = (tcls == tgt[None, :]).astype(jnp.float32)
        msum = lax.dot_general(
            meansT_ref[...], m, (((1,), (1,)), ((), ())),
            preferred_element_type=jnp.float32)
        dup = lax.dot_general(
            jnp.ones((1, T), dtype=jnp.float32), m, (((1,), (1,)), ((), ())),
            preferred_element_type=jnp.float32)
        csum = lax.dot_general(
            c, m, (((1,), (1,)), ((), ())),
            preferred_element_type=jnp.float32)
        upd = (dup > 0.0) & (csum > 0.0)
        tail_ref[...] = jnp.where(
            upd, msum / jnp.maximum(dup, 1.0), ctail_ref[...])


def _stats(xT, target_row, target, ctailT):
    tr3 = target_row.reshape(NSTEP, 1, RB)
    tgt2 = target.reshape(1, T)
    ones_spec = pl.BlockSpec((EMB, TAILC), lambda i: (0, 0))
    meansT, tidx, tail = pl.pallas_call(
        _stats_body,
        grid=(NSTEP,),
        in_specs=[
            pl.BlockSpec((1, 1, RB), lambda i: (i, 0, 0)),
            pl.BlockSpec((1, T), lambda i: (0, 0)),
            pl.BlockSpec((EMB, RB), lambda i: (0, i)),
            ones_spec,
        ],
        out_specs=[
            pl.BlockSpec((EMB, T), lambda i: (0, 0)),
            pl.BlockSpec((1, T), lambda i: (0, 0)),
            ones_spec,
        ],
        out_shape=[
            jax.ShapeDtypeStruct((EMB, T), jnp.float32),
            jax.ShapeDtypeStruct((1, T), jnp.int32),
            jax.ShapeDtypeStruct((EMB, TAILC), jnp.float32),
        ],
        scratch_shapes=[pltpu.VMEM((EMB + 8, T), jnp.float32)],
    )(tr3, tgt2, xT, ctailT)
    return meansT, tidx, tail


def _patch_cols(buf, mT, tid, cbase, ncols):
    def group_body(g, carry):
        tvec = tid[0, pl.ds(g * 16, 16)]
        local = tvec - cbase
        mask = (local >= 0) & (local < ncols)

        @pl.when(jnp.any(mask))
        def _patch():
            safe = jnp.where(mask, local, 0)

            def row_body(r, carry3):
                val = mT[r, pl.ds(g * 16, 16)]
                row = jnp.full((16,), r, dtype=jnp.int32)
                plsc.store_scatter(buf, [row, safe], val, mask=mask)
                return carry3

            lax.fori_loop(0, EMB, row_body, 0, unroll=False)
        return carry

    lax.fori_loop(0, T // 16, group_body, 0, unroll=False)


def _sc_body(centerT_hbm, meansT_hbm, tidx_hbm, outT_hbm,
             buf0, buf1, mT, tid, flg, sr0, sr1, sw0, sw1):
    wid = lax.axis_index("s") * NC + lax.axis_index("c")
    pltpu.sync_copy(meansT_hbm, mT)
    pltpu.sync_copy(tidx_hbm, tid)
    bufs = (buf0, buf1)
    srs = (sr0, sr1)
    sws = (sw0, sw1)

    def read_desc(ci, buf, sem):
        full = pltpu.make_async_copy(
            centerT_hbm.at[:, pl.ds(ci * CHC, CHC)], buf, sem)
        mid = pltpu.make_async_copy(
            centerT_hbm.at[:, pl.ds(NFULL * CHC, MIDC)],
            buf.at[:, pl.ds(0, MIDC)], sem)
        return full, mid

    def write_desc(ci, buf, sem):
        full = pltpu.make_async_copy(
            buf, outT_hbm.at[:, pl.ds(ci * CHC, CHC)], sem)
        mid = pltpu.make_async_copy(
            buf.at[:, pl.ds(0, MIDC)],
            outT_hbm.at[:, pl.ds(NFULL * CHC, MIDC)], sem)
        return full, mid

    def start_read(k, b):
        ci = wid + k * NW
        full, mid = read_desc(ci, bufs[b], srs[b])
        pl.when(ci < NFULL)(full.start)
        pl.when(ci == NFULL)(mid.start)

    def wait_read(k, b):
        ci = wid + k * NW
        full, mid = read_desc(ci, bufs[b], srs[b])
        pl.when(ci < NFULL)(full.wait)
        pl.when(ci == NFULL)(mid.wait)

    def start_write(k, b):
        ci = wid + k * NW
        full, mid = write_desc(ci, bufs[b], sws[b])
        pl.when(ci < NFULL)(full.start)
        pl.when(ci == NFULL)(mid.start)

    def wait_write(k, b):
        ci = wid + k * NW
        full, mid = write_desc(ci, bufs[b], sws[b])
        pl.when((ci >= 0) & (ci < NFULL))(full.wait)
        pl.when(ci == NFULL)(mid.wait)

    def patch(k, b, flags_v):
        ci = wid + k * NW
        cbase = ci * CHC
        lane = lax.iota(jnp.int32, 16)
        hit = jnp.max(jnp.where(lane == k, flags_v, 0)) > 0

        @pl.when(hit & (ci < NFULL))
        def _pf():
            _patch_cols(bufs[b], mT, tid, cbase, CHC)

        @pl.when(hit & (ci == NFULL))
        def _pm():
            _patch_cols(bufs[b], mT, tid, NFULL * CHC, MIDC)

    flg[pl.ds(0, 16)] = jnp.zeros((16,), jnp.int32)
    one16 = jnp.ones((16,), jnp.int32)

    def flag_body(g, carry):
        tvec = tid[0, pl.ds(g * 16, 16)]
        tsafe = jnp.where(tvec < 0, 1 << 20, tvec)
        civ = ((tsafe >> 7) * 43691) >> 17
        mine = ((civ & (NW - 1)) == wid) & (civ < NCHUNK)
        rk = jnp.where(mine, civ >> 5, 0)
        plsc.addupdate_scatter(flg, [rk], one16, mask=mine)
        return carry

    lax.fori_loop(0, T // 16, flag_body, 0, unroll=False)
    flags_v = flg[pl.ds(0, 16)]

    start_read(0, 0)

    def pair_body(k2, carry):
        for b in (0, 1):
            k = k2 * 2 + b
            wait_read(k, b)
            wait_write(k - 1, 1 - b)
            start_read(k + 1, 1 - b)
            patch(k, b, flags_v)
            start_write(k, b)
        return carry

    lax.fori_loop(0, (KMAX + 2) // 2, pair_body, 0, unroll=False)


@functools.cache
def _sc_scatter():
    return pl.kernel(
        _sc_body,
        out_type=jax.ShapeDtypeStruct((EMB, NUM_CLASSES), jnp.float32),
        mesh=plsc.VectorSubcoreMesh(core_axis_name="c", subcore_axis_name="s"),
        scratch_types=[
            pltpu.VMEM((EMB, CHC), jnp.float32),
            pltpu.VMEM((EMB, CHC), jnp.float32),
            pltpu.VMEM((EMB, T), jnp.float32),
            pltpu.VMEM((1, T), jnp.int32),
            pltpu.VMEM((16,), jnp.int32),
            pltpu.SemaphoreType.DMA,
            pltpu.SemaphoreType.DMA,
            pltpu.SemaphoreType.DMA,
            pltpu.SemaphoreType.DMA,
        ],
        compiler_params=pltpu.CompilerParams(needs_layout_passes=False),
    )


def kernel(inputs_row, target_row, target, center):
    centerT = center.T
    meansT, tidx, tail = _stats(
        inputs_row.T, target_row, target, centerT[:, NALIGN:])
    outT = _sc_scatter()(centerT, meansT, tidx)
    outT = lax.dynamic_update_slice(outT, tail, (0, NALIGN))
    return outT.T


# speedup vs baseline: 4.8204x; 1.0617x over previous
import functools

import jax
import jax.numpy as jnp
from jax import lax
from jax.experimental import pallas as pl
from jax.experimental.pallas import tpu as pltpu
from jax.experimental.pallas import tpu_sc as plsc

NUM_CLASSES = 100000
EMB = 64
N = 16384
T = 1024

RB = 2048
NSTEP = N // RB
NC = 2
NS = 16
NW = NC * NS
CHC = 384
NALIGN = (NUM_CLASSES // 128) * 128
NFULL = NALIGN // CHC
MIDC = NALIGN - NFULL * CHC
TAILC = NUM_CLASSES - NALIGN
NCHUNK = NFULL + 1
KMAX = -(-NCHUNK // NW)


def _stats_body(tr_ref, tgt_ref, xT_ref, ctail_ref,
                meansT_ref, tidx_ref, tail_ref, acc_ref):
    i = pl.program_id(0)

    @pl.when(i == 0)
    def _init():
        acc_ref[...] = jnp.zeros_like(acc_ref)

    tr = tr_ref[0, 0, :]
    tgt = tgt_ref[0, :]
    oh = (tr[:, None] == tgt[None, :]).astype(jnp.bfloat16)
    xT = xT_ref[...].astype(jnp.bfloat16)
    xa = jnp.concatenate(
        [xT, jnp.ones((8, RB), dtype=jnp.bfloat16)], axis=0)
    acc_ref[...] += lax.dot_general(
        xa, oh, (((1,), (0,)), ((), ())), preferred_element_type=jnp.float32)

    @pl.when(i == NSTEP - 1)
    def _fin():
        c = acc_ref[EMB:EMB + 1, :]
        meansT_ref[...] = acc_ref[0:EMB, :] / jnp.maximum(c, 1.0)
        tidx_ref[...] = jnp.where(c > 0.0, tgt_ref[...], -1)
        tcls = NALIGN + lax.broadcasted_iota(jnp.int32, (TAILC, 1), 0)
        m = (tcls == tgt[None, :]).astype(jnp.float32)
        msum = lax.dot_general(
            meansT_ref[...], m, (((1,), (1,)), ((), ())),
            preferred_element_type=jnp.float32)
        dup = lax.dot_general(
            jnp.ones((1, T), dtype=jnp.float32), m, (((1,), (1,)), ((), ())),
            preferred_element_type=jnp.float32)
        csum = lax.dot_general(
            c, m, (((1,), (1,)), ((), ())),
            preferred_element_type=jnp.float32)
        upd = (dup > 0.0) & (csum > 0.0)
        tail_ref[...] = jnp.where(
            upd, msum / jnp.maximum(dup, 1.0), ctail_ref[...])


def _stats(xT, target_row, target, ctailT):
    tr3 = target_row.reshape(NSTEP, 1, RB)
    tgt2 = target.reshape(1, T)
    ones_spec = pl.BlockSpec((EMB, TAILC), lambda i: (0, 0))
    meansT, tidx, tail = pl.pallas_call(
        _stats_body,
        grid=(NSTEP,),
        in_specs=[
            pl.BlockSpec((1, 1, RB), lambda i: (i, 0, 0)),
            pl.BlockSpec((1, T), lambda i: (0, 0)),
            pl.BlockSpec((EMB, RB), lambda i: (0, i)),
            ones_spec,
        ],
        out_specs=[
            pl.BlockSpec((EMB, T), lambda i: (0, 0)),
            pl.BlockSpec((1, T), lambda i: (0, 0)),
            ones_spec,
        ],
        out_shape=[
            jax.ShapeDtypeStruct((EMB, T), jnp.float32),
            jax.ShapeDtypeStruct((1, T), jnp.int32),
            jax.ShapeDtypeStruct((EMB, TAILC), jnp.float32),
        ],
        scratch_shapes=[pltpu.VMEM((EMB + 8, T), jnp.float32)],
    )(tr3, tgt2, xT, ctailT)
    return meansT, tidx, tail


def _patch_cols(buf, mT, tid, cbase, ncols):
    def group_body(g, carry):
        tvec = tid[0, pl.ds(g * 16, 16)]
        local = tvec - cbase
        mask = (local >= 0) & (local < ncols)

        @pl.when(jnp.any(mask))
        def _patch():
            safe = jnp.where(mask, local, 0)

            def row_body(r, carry3):
                val = mT[r, pl.ds(g * 16, 16)]
                row = jnp.full((16,), r, dtype=jnp.int32)
                plsc.store_scatter(buf, [row, safe], val, mask=mask)
                return carry3

            lax.fori_loop(0, EMB, row_body, 0, unroll=False)
        return carry

    lax.fori_loop(0, T // 16, group_body, 0, unroll=False)


def _sc_body(centerT_hbm, meansT_hbm, tidx_hbm, outT_hbm,
             buf0, buf1, mT, tid, flg, sr0, sr1, sw0, sw1):
    wid = lax.axis_index("s") * NC + lax.axis_index("c")
    pltpu.sync_copy(meansT_hbm, mT)
    pltpu.sync_copy(tidx_hbm, tid)
    bufs = (buf0, buf1)
    srs = (sr0, sr1)
    sws = (sw0, sw1)

    def read_desc(ci, buf, sem):
        full = pltpu.make_async_copy(
            centerT_hbm.at[:, pl.ds(ci * CHC, CHC)], buf, sem)
        mid = pltpu.make_async_copy(
            centerT_hbm.at[:, pl.ds(NFULL * CHC, MIDC)],
            buf.at[:, pl.ds(0, MIDC)], sem)
        return full, mid

    def write_desc(ci, buf, sem):
        full = pltpu.make_async_copy(
            buf, outT_hbm.at[:, pl.ds(ci * CHC, CHC)], sem)
        mid = pltpu.make_async_copy(
            buf.at[:, pl.ds(0, MIDC)],
            outT_hbm.at[:, pl.ds(NFULL * CHC, MIDC)], sem)
        return full, mid

    def start_read(k, b):
        ci = wid + k * NW
        full, mid = read_desc(ci, bufs[b], srs[b])
        pl.when(ci < NFULL)(full.start)
        pl.when(ci == NFULL)(mid.start)

    def wait_read(k, b):
        ci = wid + k * NW
        full, mid = read_desc(ci, bufs[b], srs[b])
        pl.when(ci < NFULL)(full.wait)
        pl.when(ci == NFULL)(mid.wait)

    def start_write(k, b):
        ci = wid + k * NW
        full, mid = write_desc(ci, bufs[b], sws[b])
        pl.when(ci < NFULL)(full.start)
        pl.when(ci == NFULL)(mid.start)

    def wait_write(k, b):
        ci = wid + k * NW
        full, mid = write_desc(ci, bufs[b], sws[b])
        pl.when((ci >= 0) & (ci < NFULL))(full.wait)
        pl.when(ci == NFULL)(mid.wait)

    def patch(k, b, flags_v):
        ci = wid + k * NW
        cbase = ci * CHC
        lane = lax.iota(jnp.int32, 16)
        hit = jnp.max(jnp.where(lane == k, flags_v, 0)) > 0

        @pl.when(hit & (ci < NFULL))
        def _pf():
            _patch_cols(bufs[b], mT, tid, cbase, CHC)

        @pl.when(hit & (ci == NFULL))
        def _pm():
            _patch_cols(bufs[b], mT, tid, NFULL * CHC, MIDC)

    flg[pl.ds(0, 16)] = jnp.zeros((16,), jnp.int32)
    one16 = jnp.ones((16,), jnp.int32)

    def flag_body(g, carry):
        tvec = tid[0, pl.ds(g * 16, 16)]
        tsafe = jnp.where(tvec < 0, 1 << 20, tvec)
        civ = ((tsafe >> 7) * 43691) >> 17
        mine = ((civ & (NW - 1)) == wid) & (civ < NCHUNK)
        rk = jnp.where(mine, civ >> 5, 0)
        plsc.addupdate_scatter(flg, [rk], one16, mask=mine)
        return carry

    lax.fori_loop(0, T // 16, flag_body, 0, unroll=False)
    flags_v = flg[pl.ds(0, 16)]

    start_read(0, 0)

    def pair_body(k2, carry):
        for b in (0, 1):
            k = k2 * 2 + b
            wait_read(k, b)
            wait_write(k - 1, 1 - b)
            start_read(k + 1, 1 - b)
            patch(k, b, flags_v)
            start_write(k, b)
        return carry

    lax.fori_loop(0, (KMAX + 2) // 2, pair_body, 0, unroll=False)


@functools.cache
def _sc_scatter():
    return pl.kernel(
        _sc_body,
        out_type=jax.ShapeDtypeStruct((EMB, NUM_CLASSES), jnp.float32),
        mesh=plsc.VectorSubcoreMesh(core_axis_name="c", subcore_axis_name="s"),
        scratch_types=[
            pltpu.VMEM((EMB, CHC), jnp.float32),
            pltpu.VMEM((EMB, CHC), jnp.float32),
            pltpu.VMEM((EMB, T), jnp.float32),
            pltpu.VMEM((1, T), jnp.int32),
            pltpu.VMEM((16,), jnp.int32),
            pltpu.SemaphoreType.DMA,
            pltpu.SemaphoreType.DMA,
            pltpu.SemaphoreType.DMA,
            pltpu.SemaphoreType.DMA,
        ],
        compiler_params=pltpu.CompilerParams(needs_layout_passes=False),
    )


def kernel(inputs_row, target_row, target, center):
    centerT = center.T
    meansT, tidx, tail = _stats(
        inputs_row.T, target_row, target, centerT[:, NALIGN:])
    outT = _sc_scatter()(centerT, meansT, tidx)
    outT = lax.dynamic_update_slice(outT, tail, (0, NALIGN))
    return outT.T


# speedup vs baseline: 4.9004x; 1.0166x over previous
import functools

import jax
import jax.numpy as jnp
from jax import lax
from jax.experimental import pallas as pl
from jax.experimental.pallas import tpu as pltpu
from jax.experimental.pallas import tpu_sc as plsc

NUM_CLASSES = 100000
EMB = 64
N = 16384
T = 1024

RB = 4096
NSTEP = N // RB
NC = 2
NS = 16
NW = NC * NS
CHC = 384
NALIGN = (NUM_CLASSES // 128) * 128
NFULL = NALIGN // CHC
MIDC = NALIGN - NFULL * CHC
TAILC = NUM_CLASSES - NALIGN
NCHUNK = NFULL + 1
KMAX = -(-NCHUNK // NW)


def _stats_body(tr_ref, tgt_ref, xT_ref, ctail_ref,
                meansT_ref, tidx_ref, tail_ref, acc_ref):
    i = pl.program_id(0)

    @pl.when(i == 0)
    def _init():
        acc_ref[...] = jnp.zeros_like(acc_ref)

    tr = tr_ref[0, 0, :]
    tgt = tgt_ref[0, :]
    oh = (tr[:, None] == tgt[None, :]).astype(jnp.bfloat16)
    xT = xT_ref[...].astype(jnp.bfloat16)
    xa = jnp.concatenate(
        [xT, jnp.ones((8, RB), dtype=jnp.bfloat16)], axis=0)
    acc_ref[...] += lax.dot_general(
        xa, oh, (((1,), (0,)), ((), ())), preferred_element_type=jnp.float32)

    @pl.when(i == NSTEP - 1)
    def _fin():
        c = acc_ref[EMB:EMB + 1, :]
        meansT_ref[...] = acc_ref[0:EMB, :] / jnp.maximum(c, 1.0)
        tidx_ref[...] = jnp.where(c > 0.0, tgt_ref[...], -1)
        tcls = NALIGN + lax.broadcasted_iota(jnp.int32, (TAILC, 1), 0)
        m = (tcls == tgt[None, :]).astype(jnp.float32)
        msum = lax.dot_general(
            meansT_ref[...], m, (((1,), (1,)), ((), ())),
            preferred_element_type=jnp.float32)
        dup = lax.dot_general(
            jnp.ones((1, T), dtype=jnp.float32), m, (((1,), (1,)), ((), ())),
            preferred_element_type=jnp.float32)
        csum = lax.dot_general(
            c, m, (((1,), (1,)), ((), ())),
            preferred_element_type=jnp.float32)
        upd = (dup > 0.0) & (csum > 0.0)
        tail_ref[...] = jnp.where(
            upd, msum / jnp.maximum(dup, 1.0), ctail_ref[...])


def _stats(xT, target_row, target, ctailT):
    tr3 = target_row.reshape(NSTEP, 1, RB)
    tgt2 = target.reshape(1, T)
    ones_spec = pl.BlockSpec((EMB, TAILC), lambda i: (0, 0))
    meansT, tidx, tail = pl.pallas_call(
        _stats_body,
        grid=(NSTEP,),
        in_specs=[
            pl.BlockSpec((1, 1, RB), lambda i: (i, 0, 0)),
            pl.BlockSpec((1, T), lambda i: (0, 0)),
            pl.BlockSpec((EMB, RB), lambda i: (0, i)),
            ones_spec,
        ],
        out_specs=[
            pl.BlockSpec((EMB, T), lambda i: (0, 0)),
            pl.BlockSpec((1, T), lambda i: (0, 0)),
            ones_spec,
        ],
        out_shape=[
            jax.ShapeDtypeStruct((EMB, T), jnp.float32),
            jax.ShapeDtypeStruct((1, T), jnp.int32),
            jax.ShapeDtypeStruct((EMB, TAILC), jnp.float32),
        ],
        scratch_shapes=[pltpu.VMEM((EMB + 8, T), jnp.float32)],
    )(tr3, tgt2, xT, ctailT)
    return meansT, tidx, tail


def _patch_cols(buf, mT, tid, cbase, ncols):
    def group_body(g, carry):
        tvec = tid[0, pl.ds(g * 16, 16)]
        local = tvec - cbase
        mask = (local >= 0) & (local < ncols)

        @pl.when(jnp.any(mask))
        def _patch():
            safe = jnp.where(mask, local, 0)

            def row_body(r, carry3):
                val = mT[r, pl.ds(g * 16, 16)]
                row = jnp.full((16,), r, dtype=jnp.int32)
                plsc.store_scatter(buf, [row, safe], val, mask=mask)
                return carry3

            lax.fori_loop(0, EMB, row_body, 0, unroll=False)
        return carry

    lax.fori_loop(0, T // 16, group_body, 0, unroll=False)


def _sc_body(centerT_hbm, meansT_hbm, tidx_hbm, outT_hbm,
             buf0, buf1, mT, tid, flg, sr0, sr1, sw0, sw1):
    wid = lax.axis_index("s") * NC + lax.axis_index("c")
    pltpu.sync_copy(meansT_hbm, mT)
    pltpu.sync_copy(tidx_hbm, tid)
    bufs = (buf0, buf1)
    srs = (sr0, sr1)
    sws = (sw0, sw1)

    def read_desc(ci, buf, sem):
        full = pltpu.make_async_copy(
            centerT_hbm.at[:, pl.ds(ci * CHC, CHC)], buf, sem)
        mid = pltpu.make_async_copy(
            centerT_hbm.at[:, pl.ds(NFULL * CHC, MIDC)],
            buf.at[:, pl.ds(0, MIDC)], sem)
        return full, mid

    def write_desc(ci, buf, sem):
        full = pltpu.make_async_copy(
            buf, outT_hbm.at[:, pl.ds(ci * CHC, CHC)], sem)
        mid = pltpu.make_async_copy(
            buf.at[:, pl.ds(0, MIDC)],
            outT_hbm.at[:, pl.ds(NFULL * CHC, MIDC)], sem)
        return full, mid

    def start_read(k, b):
        ci = wid + k * NW
        full, mid = read_desc(ci, bufs[b], srs[b])
        pl.when(ci < NFULL)(full.start)
        pl.when(ci == NFULL)(mid.start)

    def wait_read(k, b):
        ci = wid + k * NW
        full, mid = read_desc(ci, bufs[b], srs[b])
        pl.when(ci < NFULL)(full.wait)
        pl.when(ci == NFULL)(mid.wait)

    def start_write(k, b):
        ci = wid + k * NW
        full, mid = write_desc(ci, bufs[b], sws[b])
        pl.when(ci < NFULL)(full.start)
        pl.when(ci == NFULL)(mid.start)

    def wait_write(k, b):
        ci = wid + k * NW
        full, mid = write_desc(ci, bufs[b], sws[b])
        pl.when((ci >= 0) & (ci < NFULL))(full.wait)
        pl.when(ci == NFULL)(mid.wait)

    def patch(k, b, flags_v):
        ci = wid + k * NW
        cbase = ci * CHC
        lane = lax.iota(jnp.int32, 16)
        hit = jnp.max(jnp.where(lane == k, flags_v, 0)) > 0

        @pl.when(hit & (ci < NFULL))
        def _pf():
            _patch_cols(bufs[b], mT, tid, cbase, CHC)

        @pl.when(hit & (ci == NFULL))
        def _pm():
            _patch_cols(bufs[b], mT, tid, NFULL * CHC, MIDC)

    flg[pl.ds(0, 16)] = jnp.zeros((16,), jnp.int32)
    one16 = jnp.ones((16,), jnp.int32)

    def flag_body(g, carry):
        tvec = tid[0, pl.ds(g * 16, 16)]
        tsafe = jnp.where(tvec < 0, 1 << 20, tvec)
        civ = ((tsafe >> 7) * 43691) >> 17
        mine = ((civ & (NW - 1)) == wid) & (civ < NCHUNK)
        rk = jnp.where(mine, civ >> 5, 0)
        plsc.addupdate_scatter(flg, [rk], one16, mask=mine)
        return carry

    lax.fori_loop(0, T // 16, flag_body, 0, unroll=False)
    flags_v = flg[pl.ds(0, 16)]

    start_read(0, 0)

    def pair_body(k2, carry):
        for b in (0, 1):
            k = k2 * 2 + b
            wait_read(k, b)
            wait_write(k - 1, 1 - b)
            start_read(k + 1, 1 - b)
            patch(k, b, flags_v)
            start_write(k, b)
        return carry

    lax.fori_loop(0, (KMAX + 2) // 2, pair_body, 0, unroll=False)


@functools.cache
def _sc_scatter():
    return pl.kernel(
        _sc_body,
        out_type=jax.ShapeDtypeStruct((EMB, NUM_CLASSES), jnp.float32),
        mesh=plsc.VectorSubcoreMesh(core_axis_name="c", subcore_axis_name="s"),
        scratch_types=[
            pltpu.VMEM((EMB, CHC), jnp.float32),
            pltpu.VMEM((EMB, CHC), jnp.float32),
            pltpu.VMEM((EMB, T), jnp.float32),
            pltpu.VMEM((1, T), jnp.int32),
            pltpu.VMEM((16,), jnp.int32),
            pltpu.SemaphoreType.DMA,
            pltpu.SemaphoreType.DMA,
            pltpu.SemaphoreType.DMA,
            pltpu.SemaphoreType.DMA,
        ],
        compiler_params=pltpu.CompilerParams(needs_layout_passes=False),
    )


def kernel(inputs_row, target_row, target, center):
    centerT = center.T
    meansT, tidx, tail = _stats(
        inputs_row.T, target_row, target, centerT[:, NALIGN:])
    outT = _sc_scatter()(centerT, meansT, tidx)
    outT = lax.dynamic_update_slice(outT, tail, (0, NALIGN))
    return outT.T


# speedup vs baseline: 5.3571x; 1.0932x over previous
import functools

import jax
import jax.numpy as jnp
from jax import lax
from jax.experimental import pallas as pl
from jax.experimental.pallas import tpu as pltpu
from jax.experimental.pallas import tpu_sc as plsc

NUM_CLASSES = 100000
EMB = 64
N = 16384
T = 1024

RB = 4096
NSTEP = N // RB
NC = 2
NS = 16
NW = NC * NS
CHC = 384
NALIGN = (NUM_CLASSES // 128) * 128
NFULL = NALIGN // CHC
MIDC = NALIGN - NFULL * CHC
TAILC = NUM_CLASSES - NALIGN
NCHUNK = NFULL + 1
KMAX = -(-NCHUNK // NW)


def _stats_body(tr_ref, tgt_ref, xT_ref, ctail_ref,
                meansT_ref, tidx_ref, tail_ref, acc_ref):
    i = pl.program_id(0)

    @pl.when(i == 0)
    def _init():
        acc_ref[...] = jnp.zeros_like(acc_ref)

    tr = tr_ref[0, 0, :]
    tgt = tgt_ref[0, :]
    oh = (tr[:, None] == tgt[None, :]).astype(jnp.bfloat16)
    xT = xT_ref[...].astype(jnp.bfloat16)
    xa = jnp.concatenate(
        [xT, jnp.ones((8, RB), dtype=jnp.bfloat16)], axis=0)
    acc_ref[...] += lax.dot_general(
        xa, oh, (((1,), (0,)), ((), ())), preferred_element_type=jnp.float32)

    @pl.when(i == NSTEP - 1)
    def _fin():
        c = acc_ref[EMB:EMB + 1, :]
        meansT_ref[...] = acc_ref[0:EMB, :] / jnp.maximum(c, 1.0)
        tidx_ref[...] = jnp.where(c > 0.0, tgt_ref[...], -1)
        tcls = NALIGN + lax.broadcasted_iota(jnp.int32, (TAILC, 1), 0)
        m = (tcls == tgt[None, :]).astype(jnp.float32)
        msum = lax.dot_general(
            meansT_ref[...], m, (((1,), (1,)), ((), ())),
            preferred_element_type=jnp.float32)
        dup = lax.dot_general(
            jnp.ones((1, T), dtype=jnp.float32), m, (((1,), (1,)), ((), ())),
            preferred_element_type=jnp.float32)
        csum = lax.dot_general(
            c, m, (((1,), (1,)), ((), ())),
            preferred_element_type=jnp.float32)
        upd = (dup > 0.0) & (csum > 0.0)
        tail_ref[...] = jnp.where(
            upd, msum / jnp.maximum(dup, 1.0), ctail_ref[...])


def _stats(xT, target_row, target, ctailT):
    tr3 = target_row.reshape(NSTEP, 1, RB)
    tgt2 = target.reshape(1, T)
    ones_spec = pl.BlockSpec((EMB, TAILC), lambda i: (0, 0))
    meansT, tidx, tail = pl.pallas_call(
        _stats_body,
        grid=(NSTEP,),
        in_specs=[
            pl.BlockSpec((1, 1, RB), lambda i: (i, 0, 0)),
            pl.BlockSpec((1, T), lambda i: (0, 0)),
            pl.BlockSpec((EMB, RB), lambda i: (0, i)),
            ones_spec,
        ],
        out_specs=[
            pl.BlockSpec((EMB, T), lambda i: (0, 0)),
            pl.BlockSpec((1, T), lambda i: (0, 0)),
            ones_spec,
        ],
        out_shape=[
            jax.ShapeDtypeStruct((EMB, T), jnp.float32),
            jax.ShapeDtypeStruct((1, T), jnp.int32),
            jax.ShapeDtypeStruct((EMB, TAILC), jnp.float32),
        ],
        scratch_shapes=[pltpu.VMEM((EMB + 8, T), jnp.float32)],
    )(tr3, tgt2, xT, ctailT)
    return meansT, tidx, tail


def _patch_cols(buf, mT, tid, cbase, ncols):
    def group_body(g, carry):
        tvec = tid[0, pl.ds(g * 16, 16)]
        local = tvec - cbase
        mask = (local >= 0) & (local < ncols)

        @pl.when(jnp.any(mask))
        def _patch():
            safe = jnp.where(mask, local, 0)

            def row_body(r, carry3):
                val = mT[r, pl.ds(g * 16, 16)]
                row = jnp.full((16,), r, dtype=jnp.int32)
                plsc.store_scatter(buf, [row, safe], val, mask=mask)
                return carry3

            lax.fori_loop(0, EMB, row_body, 0, unroll=False)
        return carry

    lax.fori_loop(0, T // 16, group_body, 0, unroll=False)


def _sc_body(centerT_hbm, meansT_hbm, tidx_hbm, outT_hbm,
             buf0, buf1, mT, mTs, tid, flg, sr0, sr1, sw0, sw1):
    sid = lax.axis_index("s")
    wid = sid * NC + lax.axis_index("c")
    bufs = (buf0, buf1)
    srs = (sr0, sr1)
    sws = (sw0, sw1)

    def read_desc(ci, buf, sem):
        full = pltpu.make_async_copy(
            centerT_hbm.at[:, pl.ds(ci * CHC, CHC)], buf, sem)
        mid = pltpu.make_async_copy(
            centerT_hbm.at[:, pl.ds(NFULL * CHC, MIDC)],
            buf.at[:, pl.ds(0, MIDC)], sem)
        return full, mid

    def write_desc(ci, buf, sem):
        full = pltpu.make_async_copy(
            buf, outT_hbm.at[:, pl.ds(ci * CHC, CHC)], sem)
        mid = pltpu.make_async_copy(
            buf.at[:, pl.ds(0, MIDC)],
            outT_hbm.at[:, pl.ds(NFULL * CHC, MIDC)], sem)
        return full, mid

    def start_read(k, b):
        ci = wid + k * NW
        full, mid = read_desc(ci, bufs[b], srs[b])
        pl.when(ci < NFULL)(full.start)
        pl.when(ci == NFULL)(mid.start)

    def wait_read(k, b):
        ci = wid + k * NW
        full, mid = read_desc(ci, bufs[b], srs[b])
        pl.when(ci < NFULL)(full.wait)
        pl.when(ci == NFULL)(mid.wait)

    def start_write(k, b):
        ci = wid + k * NW
        full, mid = write_desc(ci, bufs[b], sws[b])
        pl.when(ci < NFULL)(full.start)
        pl.when(ci == NFULL)(mid.start)

    def wait_write(k, b):
        ci = wid + k * NW
        full, mid = write_desc(ci, bufs[b], sws[b])
        pl.when((ci >= 0) & (ci < NFULL))(full.wait)
        pl.when(ci == NFULL)(mid.wait)

    def patch(k, b, flags_v):
        ci = wid + k * NW
        cbase = ci * CHC
        lane = lax.iota(jnp.int32, 16)
        hit = jnp.max(jnp.where(lane == k, flags_v, 0)) > 0

        @pl.when(hit & (ci < NFULL))
        def _pf():
            _patch_cols(bufs[b], mT, tid, cbase, CHC)

        @pl.when(hit & (ci == NFULL))
        def _pm():
            _patch_cols(bufs[b], mT, tid, NFULL * CHC, MIDC)

    start_read(0, 0)

    @pl.when(sid < 8)
    def _stage():
        pltpu.sync_copy(meansT_hbm.at[pl.ds(sid * 8, 8)],
                        mTs.at[pl.ds(sid * 8, 8)])
    pltpu.sync_copy(tidx_hbm, tid)

    flg[pl.ds(0, 16)] = jnp.zeros((16,), jnp.int32)
    one16 = jnp.ones((16,), jnp.int32)

    def flag_body(g, carry):
        tvec = tid[0, pl.ds(g * 16, 16)]
        tsafe = jnp.where(tvec < 0, 1 << 20, tvec)
        civ = ((tsafe >> 7) * 43691) >> 17
        mine = ((civ & (NW - 1)) == wid) & (civ < NCHUNK)
        rk = jnp.where(mine, civ >> 5, 0)
        plsc.addupdate_scatter(flg, [rk], one16, mask=mine)
        return carry

    lax.fori_loop(0, T // 16, flag_body, 0, unroll=False)
    flags_v = flg[pl.ds(0, 16)]

    plsc.subcore_barrier()
    pltpu.sync_copy(mTs, mT)

    def pair_body(k2, carry):
        for b in (0, 1):
            k = k2 * 2 + b
            wait_read(k, b)
            wait_write(k - 1, 1 - b)
            start_read(k + 1, 1 - b)
            patch(k, b, flags_v)
            start_write(k, b)
        return carry

    lax.fori_loop(0, (KMAX + 2) // 2, pair_body, 0, unroll=False)


@functools.cache
def _sc_scatter():
    return pl.kernel(
        _sc_body,
        out_type=jax.ShapeDtypeStruct((EMB, NUM_CLASSES), jnp.float32),
        mesh=plsc.VectorSubcoreMesh(core_axis_name="c", subcore_axis_name="s"),
        scratch_types=[
            pltpu.VMEM((EMB, CHC), jnp.float32),
            pltpu.VMEM((EMB, CHC), jnp.float32),
            pltpu.VMEM((EMB, T), jnp.float32),
            pltpu.VMEM_SHARED((EMB, T), jnp.float32),
            pltpu.VMEM((1, T), jnp.int32),
            pltpu.VMEM((16,), jnp.int32),
            pltpu.SemaphoreType.DMA,
            pltpu.SemaphoreType.DMA,
            pltpu.SemaphoreType.DMA,
            pltpu.SemaphoreType.DMA,
        ],
        compiler_params=pltpu.CompilerParams(needs_layout_passes=False),
    )


def kernel(inputs_row, target_row, target, center):
    centerT = center.T
    meansT, tidx, tail = _stats(
        inputs_row.T, target_row, target, centerT[:, NALIGN:])
    outT = _sc_scatter()(centerT, meansT, tidx)
    outT = lax.dynamic_update_slice(outT, tail, (0, NALIGN))
    return outT.T
